# baseline (speedup 1.0000x reference)
import math

import jax
import jax.numpy as jnp
from jax import lax
from jax.experimental import pallas as pl
from jax.experimental.pallas import tpu as pltpu
from jax.experimental.pallas import tpu_sc as plsc

N = 10000
E = 320000
D = 128
T = 64
H = 4
DH = 16
NE = 4

NC = 2
NS = 16
NW = NC * NS

PG = 2

CB = 40
GF = 5
SCB = 40
ZW = 128

BE = 1600
BN = 1000


def _sc_gather(table, src_sl):
    es = src_sl.shape[0]
    ew = es // NW
    ni = ew // CB
    no = ni // GF
    assert ew % 8 == 0 and ew % CB == 0 and ni % GF == 0

    def body(table_hbm, src_hbm, out_hbm, idx_all, rows_v,
             idx_sem, gat_sem, out_sem):
        cid = lax.axis_index("c")
        sid = lax.axis_index("s")
        wid = sid * NC + cid
        wbase = wid * ew

        pltpu.async_copy(src_hbm.at[pl.ds(wbase, ew)], idx_all, idx_sem).wait()

        def outer(o, carry):
            g = lax.rem(o, 2)
            s0 = g * GF

            @pl.when(o >= 1)
            def _():
                for b in range(GF):
                    pltpu.make_async_copy(
                        rows_v.at[b], out_hbm.at[pl.ds(wbase, CB)],
                        out_sem).wait()
            for b in range(GF):
                ci = o * GF + b
                pltpu.async_copy(
                    table_hbm.at[idx_all.at[pl.ds(ci * CB, CB)]],
                    rows_v.at[s0 + b], gat_sem)
            for b in range(GF):
                pltpu.make_async_copy(
                    table_hbm.at[idx_all.at[pl.ds(0, CB)]], rows_v.at[s0 + b],
                    gat_sem).wait()
            for b in range(GF):
                ci = o * GF + b
                pltpu.async_copy(rows_v.at[s0 + b],
                                 out_hbm.at[pl.ds(wbase + ci * CB, CB)],
                                 out_sem)
            return carry

        lax.fori_loop(0, no, outer, 0)
        for b in range(GF):
            pltpu.make_async_copy(
                rows_v.at[b], out_hbm.at[pl.ds(wbase, CB)], out_sem).wait()

    mesh = plsc.VectorSubcoreMesh(core_axis_name="c", subcore_axis_name="s")
    f = pl.kernel(
        body,
        out_type=jax.ShapeDtypeStruct((es, D), jnp.float32),
        mesh=mesh,
        scratch_types=[
            pltpu.VMEM((ew,), jnp.int32),
            pltpu.VMEM((2 * GF, CB, D), jnp.float32),
            pltpu.SemaphoreType.DMA,
            pltpu.SemaphoreType.DMA,
            pltpu.SemaphoreType.DMA,
        ],
    )
    return f(table, src_sl)


def _edge_body(g_ref, e_ref, wfc_ref, bfc_ref, wv_ref, a_ref, r_ref, s64_ref,
               z_ref):
    msg = jnp.maximum(g_ref[...] + e_ref[...], 0.0)
    p = jnp.dot(msg, wfc_ref[...], preferred_element_type=jnp.float32)
    p = p + bfc_ref[...]
    v = jnp.dot(p, wv_ref[...], preferred_element_type=jnp.float32)
    s = jnp.clip(jnp.dot(p, a_ref[...], preferred_element_type=jnp.float32),
                 -60.0, 60.0)
    ex = jnp.exp(s)
    exrep = jnp.dot(ex, r_ref[...], preferred_element_type=jnp.float32)
    vex = v * exrep
    extail = jnp.dot(ex, s64_ref[...], preferred_element_type=jnp.float32)
    z_ref[...] = jnp.concatenate([vex, extail], axis=1)


def _tc_edge(gathered, edge_emb, wfc, bfc, wv, a, r, s64):
    es = gathered.shape[0]
    return pl.pallas_call(
        _edge_body,
        grid=(es // BE,),
        in_specs=[
            pl.BlockSpec((BE, D), lambda i: (i, 0)),
            pl.BlockSpec((BE, D), lambda i: (i, 0)),
            pl.BlockSpec((D, T), lambda i: (0, 0)),
            pl.BlockSpec((1, T), lambda i: (0, 0)),
            pl.BlockSpec((T, T), lambda i: (0, 0)),
            pl.BlockSpec((T, H), lambda i: (0, 0)),
            pl.BlockSpec((H, T), lambda i: (0, 0)),
            pl.BlockSpec((H, T), lambda i: (0, 0)),
        ],
        out_specs=pl.BlockSpec((BE, ZW), lambda i: (i, 0)),
        out_shape=jax.ShapeDtypeStruct((es, ZW), jnp.float32),
    )(gathered, edge_emb, wfc, bfc, wv, a, r, s64)


def _sc_scatter(z_sl, dst_sl, zz):
    es = z_sl.shape[0]
    ew = es // NW
    sni = ew // SCB
    assert ew % 8 == 0 and ew % SCB == 0

    def body(z_hbm, dst2_hbm, zz_hbm, out0_hbm, out1_hbm,
             idx2_v, z_v, acc_sh, ld_sem, sc_sem):
        cid = lax.axis_index("c")
        sid = lax.axis_index("s")
        wid = sid * NC + cid
        wbase = wid * ew

        pltpu.async_copy(dst2_hbm.at[wid], idx2_v, ld_sem).wait()

        @pl.when(sid == 0)
        def _():
            pltpu.sync_copy(zz_hbm, acc_sh)

        plsc.subcore_barrier()

        pltpu.async_copy(z_hbm.at[pl.ds(wbase, SCB)], z_v.at[0], ld_sem)

        def outer(o, carry):
            g = lax.rem(o, 2)

            @pl.when(o >= 1)
            def _():
                pltpu.make_async_copy(
                    z_v.at[0], acc_sh.at[idx2_v.at[0]], sc_sem).wait()
            pltpu.make_async_copy(
                z_hbm.at[pl.ds(wbase, SCB)], z_v.at[0], ld_sem).wait()
            pltpu.async_copy(z_v.at[g], acc_sh.at[idx2_v.at[o]],
                             sc_sem, add=True)

            @pl.when(o < sni - 1)
            def _():
                pltpu.async_copy(z_hbm.at[pl.ds(wbase + (o + 1) * SCB, SCB)],
                                 z_v.at[1 - g], ld_sem)
            return carry

        lax.fori_loop(0, sni, outer, 0)
        pltpu.make_async_copy(
            z_v.at[0], acc_sh.at[idx2_v.at[0]], sc_sem).wait()

        plsc.subcore_barrier()

        @pl.when(jnp.logical_and(sid == 0, cid == 0))
        def _():
            pltpu.sync_copy(acc_sh, out0_hbm)

        @pl.when(jnp.logical_and(sid == 0, cid == 1))
        def _():
            pltpu.sync_copy(acc_sh, out1_hbm)

    mesh = plsc.VectorSubcoreMesh(core_axis_name="c", subcore_axis_name="s")
    f = pl.kernel(
        body,
        out_type=(jax.ShapeDtypeStruct((N, ZW), jnp.float32),
                  jax.ShapeDtypeStruct((N, ZW), jnp.float32)),
        mesh=mesh,
        scratch_types=[
            pltpu.VMEM((sni, SCB), jnp.int32),
            pltpu.VMEM((2, SCB, ZW), jnp.float32),
            pltpu.VMEM_SHARED((N, ZW), jnp.float32),
            pltpu.SemaphoreType.DMA,
            pltpu.SemaphoreType.DMA,
        ],
    )
    return f(z_sl, dst_sl.reshape(NW, sni, SCB), zz)


def _make_node_body(nparts):
    def body(*refs):
        part_refs = refs[:nparts]
        m_ref, wo_ref, wg_ref, wec_ref, r4_ref, k4_ref, out_ref = refs[nparts:]
        acc = part_refs[0][...]
        for pr in part_refs[1:]:
            acc = acc + pr[...]
        denrep = jnp.dot(acc, m_ref[...], preferred_element_type=jnp.float32)
        vex = acc[:, :T]
        agg = vex / (denrep + 1e-9)
        attn = jnp.dot(agg, wo_ref[...], preferred_element_type=jnp.float32)
        gl = jnp.dot(attn, wg_ref[...], preferred_element_type=jnp.float32)
        gm = jnp.max(gl, axis=1, keepdims=True)
        ge = jnp.exp(gl - gm)
        gate = ge / jnp.sum(ge, axis=1, keepdims=True)
        expf = jnp.dot(attn, wec_ref[...], preferred_element_type=jnp.float32)
        grep = jnp.dot(gate, r4_ref[...], preferred_element_type=jnp.float32)
        moe = jnp.dot(grep * expf, k4_ref[...],
                      preferred_element_type=jnp.float32)
        out_ref[...] = 1.0 / (1.0 + jnp.exp(-moe))
    return body


def _tc_node(parts, m, wo, wg, wec, r4, k4):
    nparts = len(parts)
    return pl.pallas_call(
        _make_node_body(nparts),
        grid=(N // BN,),
        in_specs=(
            [pl.BlockSpec((BN, ZW), lambda i: (i, 0)) for _ in range(nparts)]
            + [
                pl.BlockSpec((ZW, T), lambda i: (0, 0)),
                pl.BlockSpec((T, T), lambda i: (0, 0)),
                pl.BlockSpec((T, NE), lambda i: (0, 0)),
                pl.BlockSpec((T, NE * T), lambda i: (0, 0)),
                pl.BlockSpec((NE, NE * T), lambda i: (0, 0)),
                pl.BlockSpec((NE * T, T), lambda i: (0, 0)),
            ]
        ),
        out_specs=pl.BlockSpec((BN, T), lambda i: (i, 0)),
        out_shape=jax.ShapeDtypeStruct((N, T), jnp.float32),
    )(*parts, m, wo, wg, wec, r4, k4)


def kernel(src_embedding, edge_index, edge_embedding, W_fc, b_fc, q, Wk, Wv,
           Wo, Wg, We):
    f32 = jnp.float32
    src = edge_index[0].astype(jnp.int32)
    dst = edge_index[1].astype(jnp.int32)

    A = (Wk.reshape(T, H, DH) * q[None, :, :]).sum(-1) * (1.0 / math.sqrt(DH))
    R = jnp.kron(jnp.eye(H, dtype=f32), jnp.ones((1, DH), f32))
    S64 = jnp.eye(H, T, dtype=f32)
    M = jnp.concatenate(
        [jnp.zeros((T, T), f32), R, jnp.zeros((ZW - T - H, T), f32)],
        axis=0)
    WeC = We.transpose(1, 0, 2).reshape(T, NE * T)
    R4 = jnp.kron(jnp.eye(NE, dtype=f32), jnp.ones((1, T), f32))
    K4 = jnp.tile(jnp.eye(T, dtype=f32), (NE, 1))
    zz = jnp.zeros((N, ZW), f32)
    bfc = b_fc.reshape(1, T)

    eg = E // PG
    parts = []
    for i in range(PG):
        src_i = lax.slice(src, (i * eg,), ((i + 1) * eg,))
        emb_i = lax.slice(edge_embedding, (i * eg, 0), ((i + 1) * eg, D))
        dst_i = lax.slice(dst, (i * eg,), ((i + 1) * eg,))
        g_i = _sc_gather(src_embedding, src_i)
        z_i = _tc_edge(g_i, emb_i, W_fc, bfc, Wv, A, R, S64)
        p0, p1 = _sc_scatter(z_i, dst_i, zz)
        parts.extend([p0, p1])

    return _tc_node(parts, M, Wo, Wg, WeC, R4, K4)


# speedup vs baseline: 1.1876x; 1.1876x over previous
import math

import jax
import jax.numpy as jnp
from jax import lax
from jax.experimental import pallas as pl
from jax.experimental.pallas import tpu as pltpu
from jax.experimental.pallas import tpu_sc as plsc

N = 10000
E = 320000
D = 128
T = 64
H = 4
DH = 16
NE = 4

NC = 2
NS = 16
NW = NC * NS

PG = 1

CB = 80
GF = 5
SCB = 80
ZW = 128

BE = 1600
BN = 1000


def _sc_gather(table, src_sl):
    es = src_sl.shape[0]
    ew = es // NW
    ni = ew // CB
    no = ni // GF
    assert ew % 8 == 0 and ew % CB == 0 and ni % GF == 0

    def body(table_hbm, src_hbm, out_hbm, idx_all, rows_v,
             idx_sem, gat_sem, out_sem):
        cid = lax.axis_index("c")
        sid = lax.axis_index("s")
        wid = sid * NC + cid
        wbase = wid * ew

        pltpu.async_copy(src_hbm.at[pl.ds(wbase, ew)], idx_all, idx_sem).wait()

        for b in range(GF):
            pltpu.async_copy(
                table_hbm.at[idx_all.at[pl.ds(b * CB, CB)]],
                rows_v.at[b], gat_sem)

        def outer(o, carry):
            g = lax.rem(o, 2)
            s0 = g * GF

            @pl.when(o >= 1)
            def _():
                for b in range(GF):
                    pltpu.make_async_copy(
                        rows_v.at[b], out_hbm.at[pl.ds(wbase, CB)],
                        out_sem).wait()

            @pl.when(o < no - 1)
            def _():
                for b in range(GF):
                    ci = (o + 1) * GF + b
                    pltpu.async_copy(
                        table_hbm.at[idx_all.at[pl.ds(ci * CB, CB)]],
                        rows_v.at[(1 - g) * GF + b], gat_sem)
            for b in range(GF):
                pltpu.make_async_copy(
                    table_hbm.at[idx_all.at[pl.ds(0, CB)]], rows_v.at[s0 + b],
                    gat_sem).wait()
            for b in range(GF):
                ci = o * GF + b
                pltpu.async_copy(rows_v.at[s0 + b],
                                 out_hbm.at[pl.ds(wbase + ci * CB, CB)],
                                 out_sem)
            return carry

        lax.fori_loop(0, no, outer, 0)
        for b in range(GF):
            pltpu.make_async_copy(
                rows_v.at[b], out_hbm.at[pl.ds(wbase, CB)], out_sem).wait()

    mesh = plsc.VectorSubcoreMesh(core_axis_name="c", subcore_axis_name="s")
    f = pl.kernel(
        body,
        out_type=jax.ShapeDtypeStruct((es, D), jnp.float32),
        mesh=mesh,
        scratch_types=[
            pltpu.VMEM((ew,), jnp.int32),
            pltpu.VMEM((2 * GF, CB, D), jnp.float32),
            pltpu.SemaphoreType.DMA,
            pltpu.SemaphoreType.DMA,
            pltpu.SemaphoreType.DMA,
        ],
    )
    return f(table, src_sl)


def _edge_body(g_ref, e_ref, wfc_ref, bfc_ref, wv_ref, a_ref, r_ref, s64_ref,
               z_ref):
    msg = jnp.maximum(g_ref[...] + e_ref[...], 0.0)
    p = jnp.dot(msg, wfc_ref[...], preferred_element_type=jnp.float32)
    p = p + bfc_ref[...]
    v = jnp.dot(p, wv_ref[...], preferred_element_type=jnp.float32)
    s = jnp.clip(jnp.dot(p, a_ref[...], preferred_element_type=jnp.float32),
                 -60.0, 60.0)
    ex = jnp.exp(s)
    exrep = jnp.dot(ex, r_ref[...], preferred_element_type=jnp.float32)
    vex = v * exrep
    extail = jnp.dot(ex, s64_ref[...], preferred_element_type=jnp.float32)
    z_ref[...] = jnp.concatenate([vex, extail], axis=1)


def _tc_edge(gathered, edge_emb, wfc, bfc, wv, a, r, s64):
    es = gathered.shape[0]
    return pl.pallas_call(
        _edge_body,
        grid=(es // BE,),
        in_specs=[
            pl.BlockSpec((BE, D), lambda i: (i, 0)),
            pl.BlockSpec((BE, D), lambda i: (i, 0)),
            pl.BlockSpec((D, T), lambda i: (0, 0)),
            pl.BlockSpec((1, T), lambda i: (0, 0)),
            pl.BlockSpec((T, T), lambda i: (0, 0)),
            pl.BlockSpec((T, H), lambda i: (0, 0)),
            pl.BlockSpec((H, T), lambda i: (0, 0)),
            pl.BlockSpec((H, T), lambda i: (0, 0)),
        ],
        out_specs=pl.BlockSpec((BE, ZW), lambda i: (i, 0)),
        out_shape=jax.ShapeDtypeStruct((es, ZW), jnp.float32),
    )(gathered, edge_emb, wfc, bfc, wv, a, r, s64)


def _sc_scatter(z_sl, dst_sl, zz):
    es = z_sl.shape[0]
    ew = es // NW
    sni = ew // SCB
    assert ew % 8 == 0 and ew % SCB == 0

    def body(z_hbm, dst2_hbm, zz_hbm, out0_hbm, out1_hbm,
             idx2_v, z_v, acc_sh, ld_sem, sc_sem):
        cid = lax.axis_index("c")
        sid = lax.axis_index("s")
        wid = sid * NC + cid
        wbase = wid * ew

        pltpu.async_copy(dst2_hbm.at[wid], idx2_v, ld_sem).wait()

        @pl.when(sid == 0)
        def _():
            pltpu.sync_copy(zz_hbm, acc_sh)

        plsc.subcore_barrier()

        pltpu.async_copy(z_hbm.at[pl.ds(wbase, SCB)], z_v.at[0], ld_sem)

        def outer(o, carry):
            g = lax.rem(o, 2)

            @pl.when(o >= 1)
            def _():
                pltpu.make_async_copy(
                    z_v.at[0], acc_sh.at[idx2_v.at[0]], sc_sem).wait()

            @pl.when(o < sni - 1)
            def _():
                pltpu.async_copy(z_hbm.at[pl.ds(wbase + (o + 1) * SCB, SCB)],
                                 z_v.at[1 - g], ld_sem)
            pltpu.make_async_copy(
                z_hbm.at[pl.ds(wbase, SCB)], z_v.at[0], ld_sem).wait()
            pltpu.async_copy(z_v.at[g], acc_sh.at[idx2_v.at[o]],
                             sc_sem, add=True)
            return carry

        lax.fori_loop(0, sni, outer, 0)
        pltpu.make_async_copy(
            z_v.at[0], acc_sh.at[idx2_v.at[0]], sc_sem).wait()

        plsc.subcore_barrier()

        @pl.when(jnp.logical_and(sid == 0, cid == 0))
        def _():
            pltpu.sync_copy(acc_sh, out0_hbm)

        @pl.when(jnp.logical_and(sid == 0, cid == 1))
        def _():
            pltpu.sync_copy(acc_sh, out1_hbm)

    mesh = plsc.VectorSubcoreMesh(core_axis_name="c", subcore_axis_name="s")
    f = pl.kernel(
        body,
        out_type=(jax.ShapeDtypeStruct((N, ZW), jnp.float32),
                  jax.ShapeDtypeStruct((N, ZW), jnp.float32)),
        mesh=mesh,
        scratch_types=[
            pltpu.VMEM((sni, SCB), jnp.int32),
            pltpu.VMEM((2, SCB, ZW), jnp.float32),
            pltpu.VMEM_SHARED((N, ZW), jnp.float32),
            pltpu.SemaphoreType.DMA,
            pltpu.SemaphoreType.DMA,
        ],
    )
    return f(z_sl, dst_sl.reshape(NW, sni, SCB), zz)


def _make_node_body(nparts):
    def body(*refs):
        part_refs = refs[:nparts]
        m_ref, wo_ref, wg_ref, wec_ref, r4_ref, k4_ref, out_ref = refs[nparts:]
        acc = part_refs[0][...]
        for pr in part_refs[1:]:
            acc = acc + pr[...]
        denrep = jnp.dot(acc, m_ref[...], preferred_element_type=jnp.float32)
        vex = acc[:, :T]
        agg = vex / (denrep + 1e-9)
        attn = jnp.dot(agg, wo_ref[...], preferred_element_type=jnp.float32)
        gl = jnp.dot(attn, wg_ref[...], preferred_element_type=jnp.float32)
        gm = jnp.max(gl, axis=1, keepdims=True)
        ge = jnp.exp(gl - gm)
        gate = ge / jnp.sum(ge, axis=1, keepdims=True)
        expf = jnp.dot(attn, wec_ref[...], preferred_element_type=jnp.float32)
        grep = jnp.dot(gate, r4_ref[...], preferred_element_type=jnp.float32)
        moe = jnp.dot(grep * expf, k4_ref[...],
                      preferred_element_type=jnp.float32)
        out_ref[...] = 1.0 / (1.0 + jnp.exp(-moe))
    return body


def _tc_node(parts, m, wo, wg, wec, r4, k4):
    nparts = len(parts)
    return pl.pallas_call(
        _make_node_body(nparts),
        grid=(N // BN,),
        in_specs=(
            [pl.BlockSpec((BN, ZW), lambda i: (i, 0)) for _ in range(nparts)]
            + [
                pl.BlockSpec((ZW, T), lambda i: (0, 0)),
                pl.BlockSpec((T, T), lambda i: (0, 0)),
                pl.BlockSpec((T, NE), lambda i: (0, 0)),
                pl.BlockSpec((T, NE * T), lambda i: (0, 0)),
                pl.BlockSpec((NE, NE * T), lambda i: (0, 0)),
                pl.BlockSpec((NE * T, T), lambda i: (0, 0)),
            ]
        ),
        out_specs=pl.BlockSpec((BN, T), lambda i: (i, 0)),
        out_shape=jax.ShapeDtypeStruct((N, T), jnp.float32),
    )(*parts, m, wo, wg, wec, r4, k4)


def kernel(src_embedding, edge_index, edge_embedding, W_fc, b_fc, q, Wk, Wv,
           Wo, Wg, We):
    f32 = jnp.float32
    src = edge_index[0].astype(jnp.int32)
    dst = edge_index[1].astype(jnp.int32)

    A = (Wk.reshape(T, H, DH) * q[None, :, :]).sum(-1) * (1.0 / math.sqrt(DH))
    R = jnp.kron(jnp.eye(H, dtype=f32), jnp.ones((1, DH), f32))
    S64 = jnp.eye(H, T, dtype=f32)
    M = jnp.concatenate(
        [jnp.zeros((T, T), f32), R, jnp.zeros((ZW - T - H, T), f32)],
        axis=0)
    WeC = We.transpose(1, 0, 2).reshape(T, NE * T)
    R4 = jnp.kron(jnp.eye(NE, dtype=f32), jnp.ones((1, T), f32))
    K4 = jnp.tile(jnp.eye(T, dtype=f32), (NE, 1))
    zz = jnp.zeros((N, ZW), f32)
    bfc = b_fc.reshape(1, T)

    eg = E // PG
    parts = []
    for i in range(PG):
        src_i = lax.slice(src, (i * eg,), ((i + 1) * eg,))
        emb_i = lax.slice(edge_embedding, (i * eg, 0), ((i + 1) * eg, D))
        dst_i = lax.slice(dst, (i * eg,), ((i + 1) * eg,))
        g_i = _sc_gather(src_embedding, src_i)
        z_i = _tc_edge(g_i, emb_i, W_fc, bfc, Wv, A, R, S64)
        p0, p1 = _sc_scatter(z_i, dst_i, zz)
        parts.extend([p0, p1])

    return _tc_node(parts, M, Wo, Wg, WeC, R4, K4)


# speedup vs baseline: 1.3597x; 1.1449x over previous
import math

import jax
import jax.numpy as jnp
from jax import lax
from jax.experimental import pallas as pl
from jax.experimental.pallas import tpu as pltpu
from jax.experimental.pallas import tpu_sc as plsc

N = 10000
E = 320000
D = 128
T = 64
H = 4
DH = 16
NE = 4

NC = 2
NS = 16
NW = NC * NS

PG = 1

CB = 80
GF = 5
SCB = 80
ZW = 128

BE = 3200
BN = 2000


def _sc_gather(table, src_sl):
    es = src_sl.shape[0]
    ew = es // NW
    ni = ew // CB
    no = ni // GF
    assert ew % 8 == 0 and ew % CB == 0 and ni % GF == 0

    def body(table_hbm, src_hbm, out_hbm, idx_all, rows_v,
             idx_sem, gat_sem, out_sem):
        cid = lax.axis_index("c")
        sid = lax.axis_index("s")
        wid = sid * NC + cid
        wbase = wid * ew

        pltpu.async_copy(src_hbm.at[pl.ds(wbase, ew)], idx_all, idx_sem).wait()

        for b in range(GF):
            pltpu.async_copy(
                table_hbm.at[idx_all.at[pl.ds(b * CB, CB)]],
                rows_v.at[b], gat_sem)

        def outer(o, carry):
            g = lax.rem(o, 2)
            s0 = g * GF

            @pl.when(o >= 1)
            def _():
                for b in range(GF):
                    pltpu.make_async_copy(
                        rows_v.at[b], out_hbm.at[pl.ds(wbase, CB)],
                        out_sem).wait()

            @pl.when(o < no - 1)
            def _():
                for b in range(GF):
                    ci = (o + 1) * GF + b
                    pltpu.async_copy(
                        table_hbm.at[idx_all.at[pl.ds(ci * CB, CB)]],
                        rows_v.at[(1 - g) * GF + b], gat_sem)
            for b in range(GF):
                pltpu.make_async_copy(
                    table_hbm.at[idx_all.at[pl.ds(0, CB)]], rows_v.at[s0 + b],
                    gat_sem).wait()
            for b in range(GF):
                ci = o * GF + b
                pltpu.async_copy(rows_v.at[s0 + b],
                                 out_hbm.at[pl.ds(wbase + ci * CB, CB)],
                                 out_sem)
            return carry

        lax.fori_loop(0, no, outer, 0)
        for b in range(GF):
            pltpu.make_async_copy(
                rows_v.at[b], out_hbm.at[pl.ds(wbase, CB)], out_sem).wait()

    mesh = plsc.VectorSubcoreMesh(core_axis_name="c", subcore_axis_name="s")
    f = pl.kernel(
        body,
        out_type=jax.ShapeDtypeStruct((es, D), jnp.float32),
        mesh=mesh,
        scratch_types=[
            pltpu.VMEM((ew,), jnp.int32),
            pltpu.VMEM((2 * GF, CB, D), jnp.float32),
            pltpu.SemaphoreType.DMA,
            pltpu.SemaphoreType.DMA,
            pltpu.SemaphoreType.DMA,
        ],
    )
    return f(table, src_sl)


def _edge_body(g_ref, e_ref, wfc_ref, bfc_ref, wv_ref, a_ref, r_ref, s64_ref,
               z_ref):
    msg = jnp.maximum(g_ref[...] + e_ref[...], 0.0)
    p = jnp.dot(msg, wfc_ref[...], preferred_element_type=jnp.float32)
    p = p + bfc_ref[...]
    v = jnp.dot(p, wv_ref[...], preferred_element_type=jnp.float32)
    s = jnp.clip(jnp.dot(p, a_ref[...], preferred_element_type=jnp.float32),
                 -60.0, 60.0)
    ex = jnp.exp(s)
    exrep = jnp.dot(ex, r_ref[...], preferred_element_type=jnp.float32)
    vex = v * exrep
    extail = jnp.dot(ex, s64_ref[...], preferred_element_type=jnp.float32)
    z_ref[...] = jnp.concatenate([vex, extail], axis=1)


def _tc_edge(gathered, edge_emb, wfc, bfc, wv, a, r, s64):
    es = gathered.shape[0]
    return pl.pallas_call(
        _edge_body,
        grid=(es // BE,),
        in_specs=[
            pl.BlockSpec((BE, D), lambda i: (i, 0)),
            pl.BlockSpec((BE, D), lambda i: (i, 0)),
            pl.BlockSpec((D, T), lambda i: (0, 0)),
            pl.BlockSpec((1, T), lambda i: (0, 0)),
            pl.BlockSpec((T, T), lambda i: (0, 0)),
            pl.BlockSpec((T, H), lambda i: (0, 0)),
            pl.BlockSpec((H, T), lambda i: (0, 0)),
            pl.BlockSpec((H, T), lambda i: (0, 0)),
        ],
        out_specs=pl.BlockSpec((BE, ZW), lambda i: (i, 0)),
        out_shape=jax.ShapeDtypeStruct((es, ZW), jnp.float32),
    )(gathered, edge_emb, wfc, bfc, wv, a, r, s64)


def _sc_scatter(z_sl, dst_sl, zz):
    es = z_sl.shape[0]
    ew = es // NW
    sni = ew // SCB
    assert ew % 8 == 0 and ew % SCB == 0

    def body(z_hbm, dst2_hbm, zz_hbm, out0_hbm, out1_hbm,
             idx2_v, z_v, acc_sh, ld_sem, sc_sem):
        cid = lax.axis_index("c")
        sid = lax.axis_index("s")
        wid = sid * NC + cid
        wbase = wid * ew

        pltpu.async_copy(dst2_hbm.at[wid], idx2_v, ld_sem).wait()

        @pl.when(sid == 0)
        def _():
            pltpu.sync_copy(zz_hbm, acc_sh)

        plsc.subcore_barrier()

        pltpu.async_copy(z_hbm.at[pl.ds(wbase, SCB)], z_v.at[0], ld_sem)

        def outer(o, carry):
            g = lax.rem(o, 2)

            @pl.when(o >= 1)
            def _():
                pltpu.make_async_copy(
                    z_v.at[0], acc_sh.at[idx2_v.at[0]], sc_sem).wait()

            @pl.when(o < sni - 1)
            def _():
                pltpu.async_copy(z_hbm.at[pl.ds(wbase + (o + 1) * SCB, SCB)],
                                 z_v.at[1 - g], ld_sem)
            pltpu.make_async_copy(
                z_hbm.at[pl.ds(wbase, SCB)], z_v.at[0], ld_sem).wait()
            pltpu.async_copy(z_v.at[g], acc_sh.at[idx2_v.at[o]],
                             sc_sem, add=True)
            return carry

        lax.fori_loop(0, sni, outer, 0)
        pltpu.make_async_copy(
            z_v.at[0], acc_sh.at[idx2_v.at[0]], sc_sem).wait()

        plsc.subcore_barrier()

        @pl.when(jnp.logical_and(sid == 0, cid == 0))
        def _():
            pltpu.sync_copy(acc_sh, out0_hbm)

        @pl.when(jnp.logical_and(sid == 0, cid == 1))
        def _():
            pltpu.sync_copy(acc_sh, out1_hbm)

    mesh = plsc.VectorSubcoreMesh(core_axis_name="c", subcore_axis_name="s")
    f = pl.kernel(
        body,
        out_type=(jax.ShapeDtypeStruct((N, ZW), jnp.float32),
                  jax.ShapeDtypeStruct((N, ZW), jnp.float32)),
        mesh=mesh,
        scratch_types=[
            pltpu.VMEM((sni, SCB), jnp.int32),
            pltpu.VMEM((2, SCB, ZW), jnp.float32),
            pltpu.VMEM_SHARED((N, ZW), jnp.float32),
            pltpu.SemaphoreType.DMA,
            pltpu.SemaphoreType.DMA,
        ],
    )
    return f(z_sl, dst_sl.reshape(NW, sni, SCB), zz)


def _make_node_body(nparts):
    def body(*refs):
        part_refs = refs[:nparts]
        m_ref, wo_ref, wg_ref, wec_ref, r4_ref, k4_ref, out_ref = refs[nparts:]
        acc = part_refs[0][...]
        for pr in part_refs[1:]:
            acc = acc + pr[...]
        denrep = jnp.dot(acc, m_ref[...], preferred_element_type=jnp.float32)
        vex = acc[:, :T]
        agg = vex / (denrep + 1e-9)
        attn = jnp.dot(agg, wo_ref[...], preferred_element_type=jnp.float32)
        gl = jnp.dot(attn, wg_ref[...], preferred_element_type=jnp.float32)
        gm = jnp.max(gl, axis=1, keepdims=True)
        ge = jnp.exp(gl - gm)
        gate = ge / jnp.sum(ge, axis=1, keepdims=True)
        expf = jnp.dot(attn, wec_ref[...], preferred_element_type=jnp.float32)
        grep = jnp.dot(gate, r4_ref[...], preferred_element_type=jnp.float32)
        moe = jnp.dot(grep * expf, k4_ref[...],
                      preferred_element_type=jnp.float32)
        out_ref[...] = 1.0 / (1.0 + jnp.exp(-moe))
    return body


def _tc_node(parts, m, wo, wg, wec, r4, k4):
    nparts = len(parts)
    return pl.pallas_call(
        _make_node_body(nparts),
        grid=(N // BN,),
        in_specs=(
            [pl.BlockSpec((BN, ZW), lambda i: (i, 0)) for _ in range(nparts)]
            + [
                pl.BlockSpec((ZW, T), lambda i: (0, 0)),
                pl.BlockSpec((T, T), lambda i: (0, 0)),
                pl.BlockSpec((T, NE), lambda i: (0, 0)),
                pl.BlockSpec((T, NE * T), lambda i: (0, 0)),
                pl.BlockSpec((NE, NE * T), lambda i: (0, 0)),
                pl.BlockSpec((NE * T, T), lambda i: (0, 0)),
            ]
        ),
        out_specs=pl.BlockSpec((BN, T), lambda i: (i, 0)),
        out_shape=jax.ShapeDtypeStruct((N, T), jnp.float32),
    )(*parts, m, wo, wg, wec, r4, k4)


def kernel(src_embedding, edge_index, edge_embedding, W_fc, b_fc, q, Wk, Wv,
           Wo, Wg, We):
    f32 = jnp.float32
    src = edge_index[0].astype(jnp.int32)
    dst = edge_index[1].astype(jnp.int32)

    A = (Wk.reshape(T, H, DH) * q[None, :, :]).sum(-1) * (1.0 / math.sqrt(DH))
    R = jnp.kron(jnp.eye(H, dtype=f32), jnp.ones((1, DH), f32))
    S64 = jnp.eye(H, T, dtype=f32)
    M = jnp.concatenate(
        [jnp.zeros((T, T), f32), R, jnp.zeros((ZW - T - H, T), f32)],
        axis=0)
    WeC = We.transpose(1, 0, 2).reshape(T, NE * T)
    R4 = jnp.kron(jnp.eye(NE, dtype=f32), jnp.ones((1, T), f32))
    K4 = jnp.tile(jnp.eye(T, dtype=f32), (NE, 1))
    zz = jnp.zeros((N, ZW), f32)
    bfc = b_fc.reshape(1, T)

    eg = E // PG
    parts = []
    for i in range(PG):
        src_i = lax.slice(src, (i * eg,), ((i + 1) * eg,))
        emb_i = lax.slice(edge_embedding, (i * eg, 0), ((i + 1) * eg, D))
        dst_i = lax.slice(dst, (i * eg,), ((i + 1) * eg,))
        g_i = _sc_gather(src_embedding, src_i)
        z_i = _tc_edge(g_i, emb_i, W_fc, bfc, Wv, A, R, S64)
        p0, p1 = _sc_scatter(z_i, dst_i, zz)
        parts.extend([p0, p1])

    return _tc_node(parts, M, Wo, Wg, WeC, R4, K4)


# speedup vs baseline: 1.4691x; 1.0805x over previous
import math

import jax
import jax.numpy as jnp
from jax import lax
from jax.experimental import pallas as pl
from jax.experimental.pallas import tpu as pltpu
from jax.experimental.pallas import tpu_sc as plsc

N = 10000
E = 320000
D = 128
T = 64
H = 4
DH = 16
NE = 4

NC = 2
NS = 16
NW = NC * NS

PG = 1

CB = 80
GF = 5
SCB = 80
ZW = 128

BE = 6400
BN = 2000


def _sc_gather(table, src_sl):
    es = src_sl.shape[0]
    ew = es // NW
    ni = ew // CB
    no = ni // GF
    assert ew % 8 == 0 and ew % CB == 0 and ni % GF == 0

    def body(table_hbm, src_hbm, out_hbm, idx_all, rows_v,
             idx_sem, gat_sem, out_sem):
        cid = lax.axis_index("c")
        sid = lax.axis_index("s")
        wid = sid * NC + cid
        wbase = wid * ew

        pltpu.async_copy(src_hbm.at[pl.ds(wbase, ew)], idx_all, idx_sem).wait()

        for b in range(GF):
            pltpu.async_copy(
                table_hbm.at[idx_all.at[pl.ds(b * CB, CB)]],
                rows_v.at[b], gat_sem)

        def outer(o, carry):
            g = lax.rem(o, 2)
            s0 = g * GF

            @pl.when(o >= 1)
            def _():
                for b in range(GF):
                    pltpu.make_async_copy(
                        rows_v.at[b], out_hbm.at[pl.ds(wbase, CB)],
                        out_sem).wait()

            @pl.when(o < no - 1)
            def _():
                for b in range(GF):
                    ci = (o + 1) * GF + b
                    pltpu.async_copy(
                        table_hbm.at[idx_all.at[pl.ds(ci * CB, CB)]],
                        rows_v.at[(1 - g) * GF + b], gat_sem)
            for b in range(GF):
                pltpu.make_async_copy(
                    table_hbm.at[idx_all.at[pl.ds(0, CB)]], rows_v.at[s0 + b],
                    gat_sem).wait()
            for b in range(GF):
                ci = o * GF + b
                pltpu.async_copy(rows_v.at[s0 + b],
                                 out_hbm.at[pl.ds(wbase + ci * CB, CB)],
                                 out_sem)
            return carry

        lax.fori_loop(0, no, outer, 0)
        for b in range(GF):
            pltpu.make_async_copy(
                rows_v.at[b], out_hbm.at[pl.ds(wbase, CB)], out_sem).wait()

    mesh = plsc.VectorSubcoreMesh(core_axis_name="c", subcore_axis_name="s")
    f = pl.kernel(
        body,
        out_type=jax.ShapeDtypeStruct((es, D), jnp.float32),
        mesh=mesh,
        scratch_types=[
            pltpu.VMEM((ew,), jnp.int32),
            pltpu.VMEM((2 * GF, CB, D), jnp.float32),
            pltpu.SemaphoreType.DMA,
            pltpu.SemaphoreType.DMA,
            pltpu.SemaphoreType.DMA,
        ],
    )
    return f(table, src_sl)


def _edge_body(g_ref, e_ref, wfc_ref, bfc_ref, wv_ref, a_ref, r_ref, s64_ref,
               z_ref):
    msg = jnp.maximum(g_ref[...] + e_ref[...], 0.0)
    p = jnp.dot(msg, wfc_ref[...], preferred_element_type=jnp.float32)
    p = p + bfc_ref[...]
    v = jnp.dot(p, wv_ref[...], preferred_element_type=jnp.float32)
    s = jnp.clip(jnp.dot(p, a_ref[...], preferred_element_type=jnp.float32),
                 -60.0, 60.0)
    ex = jnp.exp(s)
    exrep = jnp.dot(ex, r_ref[...], preferred_element_type=jnp.float32)
    vex = v * exrep
    extail = jnp.dot(ex, s64_ref[...], preferred_element_type=jnp.float32)
    z_ref[...] = jnp.concatenate([vex, extail], axis=1)


def _tc_edge(gathered, edge_emb, wfc, bfc, wv, a, r, s64):
    es = gathered.shape[0]
    return pl.pallas_call(
        _edge_body,
        grid=(es // BE,),
        in_specs=[
            pl.BlockSpec((BE, D), lambda i: (i, 0)),
            pl.BlockSpec((BE, D), lambda i: (i, 0)),
            pl.BlockSpec((D, T), lambda i: (0, 0)),
            pl.BlockSpec((1, T), lambda i: (0, 0)),
            pl.BlockSpec((T, T), lambda i: (0, 0)),
            pl.BlockSpec((T, H), lambda i: (0, 0)),
            pl.BlockSpec((H, T), lambda i: (0, 0)),
            pl.BlockSpec((H, T), lambda i: (0, 0)),
        ],
        out_specs=pl.BlockSpec((BE, ZW), lambda i: (i, 0)),
        out_shape=jax.ShapeDtypeStruct((es, ZW), jnp.float32),
    )(gathered, edge_emb, wfc, bfc, wv, a, r, s64)


def _sc_scatter(z_sl, dst_sl, zz):
    es = z_sl.shape[0]
    ew = es // NW
    sni = ew // SCB
    assert ew % 8 == 0 and ew % SCB == 0

    def body(z_hbm, dst2_hbm, zz_hbm, out0_hbm, out1_hbm,
             idx2_v, z_v, acc_sh, ld_sem, sc_sem):
        cid = lax.axis_index("c")
        sid = lax.axis_index("s")
        wid = sid * NC + cid
        wbase = wid * ew

        pltpu.async_copy(dst2_hbm.at[wid], idx2_v, ld_sem).wait()

        @pl.when(sid == 0)
        def _():
            pltpu.sync_copy(zz_hbm, acc_sh)

        plsc.subcore_barrier()

        pltpu.async_copy(z_hbm.at[pl.ds(wbase, SCB)], z_v.at[0], ld_sem)

        def outer(o, carry):
            g = lax.rem(o, 2)

            @pl.when(o >= 1)
            def _():
                pltpu.make_async_copy(
                    z_v.at[0], acc_sh.at[idx2_v.at[0]], sc_sem).wait()

            @pl.when(o < sni - 1)
            def _():
                pltpu.async_copy(z_hbm.at[pl.ds(wbase + (o + 1) * SCB, SCB)],
                                 z_v.at[1 - g], ld_sem)
            pltpu.make_async_copy(
                z_hbm.at[pl.ds(wbase, SCB)], z_v.at[0], ld_sem).wait()
            pltpu.async_copy(z_v.at[g], acc_sh.at[idx2_v.at[o]],
                             sc_sem, add=True)
            return carry

        lax.fori_loop(0, sni, outer, 0)
        pltpu.make_async_copy(
            z_v.at[0], acc_sh.at[idx2_v.at[0]], sc_sem).wait()

        plsc.subcore_barrier()

        @pl.when(jnp.logical_and(sid == 0, cid == 0))
        def _():
            pltpu.sync_copy(acc_sh, out0_hbm)

        @pl.when(jnp.logical_and(sid == 0, cid == 1))
        def _():
            pltpu.sync_copy(acc_sh, out1_hbm)

    mesh = plsc.VectorSubcoreMesh(core_axis_name="c", subcore_axis_name="s")
    f = pl.kernel(
        body,
        out_type=(jax.ShapeDtypeStruct((N, ZW), jnp.float32),
                  jax.ShapeDtypeStruct((N, ZW), jnp.float32)),
        mesh=mesh,
        scratch_types=[
            pltpu.VMEM((sni, SCB), jnp.int32),
            pltpu.VMEM((2, SCB, ZW), jnp.float32),
            pltpu.VMEM_SHARED((N, ZW), jnp.float32),
            pltpu.SemaphoreType.DMA,
            pltpu.SemaphoreType.DMA,
        ],
    )
    return f(z_sl, dst_sl.reshape(NW, sni, SCB), zz)


def _make_node_body(nparts):
    def body(*refs):
        part_refs = refs[:nparts]
        m_ref, wo_ref, wg_ref, wec_ref, r4_ref, k4_ref, out_ref = refs[nparts:]
        acc = part_refs[0][...]
        for pr in part_refs[1:]:
            acc = acc + pr[...]
        denrep = jnp.dot(acc, m_ref[...], preferred_element_type=jnp.float32)
        vex = acc[:, :T]
        agg = vex / (denrep + 1e-9)
        attn = jnp.dot(agg, wo_ref[...], preferred_element_type=jnp.float32)
        gl = jnp.dot(attn, wg_ref[...], preferred_element_type=jnp.float32)
        gm = jnp.max(gl, axis=1, keepdims=True)
        ge = jnp.exp(gl - gm)
        gate = ge / jnp.sum(ge, axis=1, keepdims=True)
        expf = jnp.dot(attn, wec_ref[...], preferred_element_type=jnp.float32)
        grep = jnp.dot(gate, r4_ref[...], preferred_element_type=jnp.float32)
        moe = jnp.dot(grep * expf, k4_ref[...],
                      preferred_element_type=jnp.float32)
        out_ref[...] = 1.0 / (1.0 + jnp.exp(-moe))
    return body


def _tc_node(parts, m, wo, wg, wec, r4, k4):
    nparts = len(parts)
    return pl.pallas_call(
        _make_node_body(nparts),
        grid=(N // BN,),
        in_specs=(
            [pl.BlockSpec((BN, ZW), lambda i: (i, 0)) for _ in range(nparts)]
            + [
                pl.BlockSpec((ZW, T), lambda i: (0, 0)),
                pl.BlockSpec((T, T), lambda i: (0, 0)),
                pl.BlockSpec((T, NE), lambda i: (0, 0)),
                pl.BlockSpec((T, NE * T), lambda i: (0, 0)),
                pl.BlockSpec((NE, NE * T), lambda i: (0, 0)),
                pl.BlockSpec((NE * T, T), lambda i: (0, 0)),
            ]
        ),
        out_specs=pl.BlockSpec((BN, T), lambda i: (i, 0)),
        out_shape=jax.ShapeDtypeStruct((N, T), jnp.float32),
    )(*parts, m, wo, wg, wec, r4, k4)


def kernel(src_embedding, edge_index, edge_embedding, W_fc, b_fc, q, Wk, Wv,
           Wo, Wg, We):
    f32 = jnp.float32
    src = edge_index[0].astype(jnp.int32)
    dst = edge_index[1].astype(jnp.int32)

    A = (Wk.reshape(T, H, DH) * q[None, :, :]).sum(-1) * (1.0 / math.sqrt(DH))
    R = jnp.kron(jnp.eye(H, dtype=f32), jnp.ones((1, DH), f32))
    S64 = jnp.eye(H, T, dtype=f32)
    M = jnp.concatenate(
        [jnp.zeros((T, T), f32), R, jnp.zeros((ZW - T - H, T), f32)],
        axis=0)
    WeC = We.transpose(1, 0, 2).reshape(T, NE * T)
    R4 = jnp.kron(jnp.eye(NE, dtype=f32), jnp.ones((1, T), f32))
    K4 = jnp.tile(jnp.eye(T, dtype=f32), (NE, 1))
    zz = jnp.zeros((N, ZW), f32)
    bfc = b_fc.reshape(1, T)

    eg = E // PG
    parts = []
    for i in range(PG):
        src_i = lax.slice(src, (i * eg,), ((i + 1) * eg,))
        emb_i = lax.slice(edge_embedding, (i * eg, 0), ((i + 1) * eg, D))
        dst_i = lax.slice(dst, (i * eg,), ((i + 1) * eg,))
        g_i = _sc_gather(src_embedding, src_i)
        z_i = _tc_edge(g_i, emb_i, W_fc, bfc, Wv, A, R, S64)
        p0, p1 = _sc_scatter(z_i, dst_i, zz)
        parts.extend([p0, p1])

    return _tc_node(parts, M, Wo, Wg, WeC, R4, K4)


# speedup vs baseline: 1.4886x; 1.0133x over previous
import math

import jax
import jax.numpy as jnp
from jax import lax
from jax.experimental import pallas as pl
from jax.experimental.pallas import tpu as pltpu
from jax.experimental.pallas import tpu_sc as plsc

N = 10000
E = 320000
D = 128
T = 64
H = 4
DH = 16
NE = 4

NC = 2
NS = 16
NW = NC * NS

PG = 1

CB = 80
GF = 5
SCB = 80
ZW = 128

BE = 8000
BN = 2000


def _sc_gather(table, src_sl):
    es = src_sl.shape[0]
    ew = es // NW
    ni = ew // CB
    no = ni // GF
    assert ew % 8 == 0 and ew % CB == 0 and ni % GF == 0

    def body(table_hbm, src_hbm, out_hbm, idx_all, rows_v,
             idx_sem, gat_sem, out_sem):
        cid = lax.axis_index("c")
        sid = lax.axis_index("s")
        wid = sid * NC + cid
        wbase = wid * ew

        pltpu.async_copy(src_hbm.at[pl.ds(wbase, ew)], idx_all, idx_sem).wait()

        for b in range(GF):
            pltpu.async_copy(
                table_hbm.at[idx_all.at[pl.ds(b * CB, CB)]],
                rows_v.at[b], gat_sem)

        def outer(o, carry):
            g = lax.rem(o, 2)
            s0 = g * GF

            @pl.when(o >= 1)
            def _():
                for b in range(GF):
                    pltpu.make_async_copy(
                        rows_v.at[b], out_hbm.at[pl.ds(wbase, CB)],
                        out_sem).wait()

            @pl.when(o < no - 1)
            def _():
                for b in range(GF):
                    ci = (o + 1) * GF + b
                    pltpu.async_copy(
                        table_hbm.at[idx_all.at[pl.ds(ci * CB, CB)]],
                        rows_v.at[(1 - g) * GF + b], gat_sem)
            for b in range(GF):
                pltpu.make_async_copy(
                    table_hbm.at[idx_all.at[pl.ds(0, CB)]], rows_v.at[s0 + b],
                    gat_sem).wait()
            for b in range(GF):
                ci = o * GF + b
                pltpu.async_copy(rows_v.at[s0 + b],
                                 out_hbm.at[pl.ds(wbase + ci * CB, CB)],
                                 out_sem)
            return carry

        lax.fori_loop(0, no, outer, 0)
        for b in range(GF):
            pltpu.make_async_copy(
                rows_v.at[b], out_hbm.at[pl.ds(wbase, CB)], out_sem).wait()

    mesh = plsc.VectorSubcoreMesh(core_axis_name="c", subcore_axis_name="s")
    f = pl.kernel(
        body,
        out_type=jax.ShapeDtypeStruct((es, D), jnp.float32),
        mesh=mesh,
        scratch_types=[
            pltpu.VMEM((ew,), jnp.int32),
            pltpu.VMEM((2 * GF, CB, D), jnp.float32),
            pltpu.SemaphoreType.DMA,
            pltpu.SemaphoreType.DMA,
            pltpu.SemaphoreType.DMA,
        ],
    )
    return f(table, src_sl)


def _edge_body(g_ref, e_ref, wfc_ref, bfc_ref, wv_ref, a_ref, r_ref, s64_ref,
               z_ref):
    msg = jnp.maximum(g_ref[...] + e_ref[...], 0.0)
    p = jnp.dot(msg, wfc_ref[...], preferred_element_type=jnp.float32)
    p = p + bfc_ref[...]
    v = jnp.dot(p, wv_ref[...], preferred_element_type=jnp.float32)
    s = jnp.clip(jnp.dot(p, a_ref[...], preferred_element_type=jnp.float32),
                 -60.0, 60.0)
    ex = jnp.exp(s)
    exrep = jnp.dot(ex, r_ref[...], preferred_element_type=jnp.float32)
    vex = v * exrep
    extail = jnp.dot(ex, s64_ref[...], preferred_element_type=jnp.float32)
    z_ref[...] = jnp.concatenate([vex, extail], axis=1)


def _tc_edge(gathered, edge_emb, wfc, bfc, wv, a, r, s64):
    es = gathered.shape[0]
    return pl.pallas_call(
        _edge_body,
        grid=(es // BE,),
        in_specs=[
            pl.BlockSpec((BE, D), lambda i: (i, 0)),
            pl.BlockSpec((BE, D), lambda i: (i, 0)),
            pl.BlockSpec((D, T), lambda i: (0, 0)),
            pl.BlockSpec((1, T), lambda i: (0, 0)),
            pl.BlockSpec((T, T), lambda i: (0, 0)),
            pl.BlockSpec((T, H), lambda i: (0, 0)),
            pl.BlockSpec((H, T), lambda i: (0, 0)),
            pl.BlockSpec((H, T), lambda i: (0, 0)),
        ],
        out_specs=pl.BlockSpec((BE, ZW), lambda i: (i, 0)),
        out_shape=jax.ShapeDtypeStruct((es, ZW), jnp.float32),
    )(gathered, edge_emb, wfc, bfc, wv, a, r, s64)


def _sc_scatter(z_sl, dst_sl, zz):
    es = z_sl.shape[0]
    ew = es // NW
    sni = ew // SCB
    assert ew % 8 == 0 and ew % SCB == 0

    def body(z_hbm, dst2_hbm, zz_hbm, out0_hbm, out1_hbm,
             idx2_v, z_v, acc_sh, ld_sem, sc_sem):
        cid = lax.axis_index("c")
        sid = lax.axis_index("s")
        wid = sid * NC + cid
        wbase = wid * ew

        pltpu.async_copy(dst2_hbm.at[wid], idx2_v, ld_sem).wait()

        @pl.when(sid == 0)
        def _():
            pltpu.sync_copy(zz_hbm, acc_sh)

        plsc.subcore_barrier()

        pltpu.async_copy(z_hbm.at[pl.ds(wbase, SCB)], z_v.at[0], ld_sem)

        def outer(o, carry):
            g = lax.rem(o, 2)

            @pl.when(o >= 1)
            def _():
                pltpu.make_async_copy(
                    z_v.at[0], acc_sh.at[idx2_v.at[0]], sc_sem).wait()

            @pl.when(o < sni - 1)
            def _():
                pltpu.async_copy(z_hbm.at[pl.ds(wbase + (o + 1) * SCB, SCB)],
                                 z_v.at[1 - g], ld_sem)
            pltpu.make_async_copy(
                z_hbm.at[pl.ds(wbase, SCB)], z_v.at[0], ld_sem).wait()
            pltpu.async_copy(z_v.at[g], acc_sh.at[idx2_v.at[o]],
                             sc_sem, add=True)
            return carry

        lax.fori_loop(0, sni, outer, 0)
        pltpu.make_async_copy(
            z_v.at[0], acc_sh.at[idx2_v.at[0]], sc_sem).wait()

        plsc.subcore_barrier()

        @pl.when(jnp.logical_and(sid == 0, cid == 0))
        def _():
            pltpu.sync_copy(acc_sh, out0_hbm)

        @pl.when(jnp.logical_and(sid == 0, cid == 1))
        def _():
            pltpu.sync_copy(acc_sh, out1_hbm)

    mesh = plsc.VectorSubcoreMesh(core_axis_name="c", subcore_axis_name="s")
    f = pl.kernel(
        body,
        out_type=(jax.ShapeDtypeStruct((N, ZW), jnp.float32),
                  jax.ShapeDtypeStruct((N, ZW), jnp.float32)),
        mesh=mesh,
        scratch_types=[
            pltpu.VMEM((sni, SCB), jnp.int32),
            pltpu.VMEM((2, SCB, ZW), jnp.float32),
            pltpu.VMEM_SHARED((N, ZW), jnp.float32),
            pltpu.SemaphoreType.DMA,
            pltpu.SemaphoreType.DMA,
        ],
    )
    return f(z_sl, dst_sl.reshape(NW, sni, SCB), zz)


def _make_node_body(nparts):
    def body(*refs):
        part_refs = refs[:nparts]
        m_ref, wo_ref, wg_ref, wec_ref, r4_ref, k4_ref, out_ref = refs[nparts:]
        acc = part_refs[0][...]
        for pr in part_refs[1:]:
            acc = acc + pr[...]
        denrep = jnp.dot(acc, m_ref[...], preferred_element_type=jnp.float32)
        vex = acc[:, :T]
        agg = vex / (denrep + 1e-9)
        attn = jnp.dot(agg, wo_ref[...], preferred_element_type=jnp.float32)
        gl = jnp.dot(attn, wg_ref[...], preferred_element_type=jnp.float32)
        gm = jnp.max(gl, axis=1, keepdims=True)
        ge = jnp.exp(gl - gm)
        gate = ge / jnp.sum(ge, axis=1, keepdims=True)
        expf = jnp.dot(attn, wec_ref[...], preferred_element_type=jnp.float32)
        grep = jnp.dot(gate, r4_ref[...], preferred_element_type=jnp.float32)
        moe = jnp.dot(grep * expf, k4_ref[...],
                      preferred_element_type=jnp.float32)
        out_ref[...] = 1.0 / (1.0 + jnp.exp(-moe))
    return body


def _tc_node(parts, m, wo, wg, wec, r4, k4):
    nparts = len(parts)
    return pl.pallas_call(
        _make_node_body(nparts),
        grid=(N // BN,),
        in_specs=(
            [pl.BlockSpec((BN, ZW), lambda i: (i, 0)) for _ in range(nparts)]
            + [
                pl.BlockSpec((ZW, T), lambda i: (0, 0)),
                pl.BlockSpec((T, T), lambda i: (0, 0)),
                pl.BlockSpec((T, NE), lambda i: (0, 0)),
                pl.BlockSpec((T, NE * T), lambda i: (0, 0)),
                pl.BlockSpec((NE, NE * T), lambda i: (0, 0)),
                pl.BlockSpec((NE * T, T), lambda i: (0, 0)),
            ]
        ),
        out_specs=pl.BlockSpec((BN, T), lambda i: (i, 0)),
        out_shape=jax.ShapeDtypeStruct((N, T), jnp.float32),
    )(*parts, m, wo, wg, wec, r4, k4)


def kernel(src_embedding, edge_index, edge_embedding, W_fc, b_fc, q, Wk, Wv,
           Wo, Wg, We):
    f32 = jnp.float32
    src = edge_index[0].astype(jnp.int32)
    dst = edge_index[1].astype(jnp.int32)

    A = (Wk.reshape(T, H, DH) * q[None, :, :]).sum(-1) * (1.0 / math.sqrt(DH))
    R = jnp.kron(jnp.eye(H, dtype=f32), jnp.ones((1, DH), f32))
    S64 = jnp.eye(H, T, dtype=f32)
    M = jnp.concatenate(
        [jnp.zeros((T, T), f32), R, jnp.zeros((ZW - T - H, T), f32)],
        axis=0)
    WeC = We.transpose(1, 0, 2).reshape(T, NE * T)
    R4 = jnp.kron(jnp.eye(NE, dtype=f32), jnp.ones((1, T), f32))
    K4 = jnp.tile(jnp.eye(T, dtype=f32), (NE, 1))
    zz = jnp.zeros((N, ZW), f32)
    bfc = b_fc.reshape(1, T)

    eg = E // PG
    parts = []
    for i in range(PG):
        src_i = lax.slice(src, (i * eg,), ((i + 1) * eg,))
        emb_i = lax.slice(edge_embedding, (i * eg, 0), ((i + 1) * eg, D))
        dst_i = lax.slice(dst, (i * eg,), ((i + 1) * eg,))
        g_i = _sc_gather(src_embedding, src_i)
        z_i = _tc_edge(g_i, emb_i, W_fc, bfc, Wv, A, R, S64)
        p0, p1 = _sc_scatter(z_i, dst_i, zz)
        parts.extend([p0, p1])

    return _tc_node(parts, M, Wo, Wg, WeC, R4, K4)
